# baseline (speedup 1.0000x reference)
import jax
import jax.numpy as jnp
from jax import lax
from jax.experimental import pallas as pl


def _body(n, x_ref, w_ref, xout_ref, score_ref):
    xv = x_ref[...]
    w = w_ref[...]
    w2 = w * lax.rsqrt(jnp.sum(w * w))
    w2t = lax.transpose(w2, (1, 0))
    d = w.shape[1]
    wb = lax.broadcast_in_dim(w2t, (d, d), (0, 1))
    sb = lax.dot_general(
        xv, wb, (((1,), (0,)), ((), ())), preferred_element_type=jnp.float32
    )
    colsum = jnp.sum(xv, axis=0, keepdims=True)
    c = jnp.sum(colsum * w2) / n
    scb = jnp.tanh(sb - c)
    xout_ref[...] = xv * scb
    sc = lax.slice(scb, (0, 0), (scb.shape[0], 1))
    score_ref[...] = lax.transpose(sc, (1, 0))


def kernel(x, edge_index, weight):
    n, d = x.shape

    def body(*refs):
        _body(n, *refs)

    x_out, score = pl.pallas_call(
        body,
        out_shape=(
            jax.ShapeDtypeStruct((n, d), x.dtype),
            jax.ShapeDtypeStruct((1, n), x.dtype),
        ),
    )(x, weight)
    return x_out, score
